# Initial kernel scaffold; baseline (speedup 1.0000x reference)
import jax
import jax.numpy as jnp
from jax.experimental import pallas as pl


def kernel(grids, W_in, b_in, gcn_W, gcn_b, ln_g, ln_b, W_out, b_out):
    raise NotImplementedError("write your pallas kernel here")


# speedup vs baseline: 9.5900x; 9.5900x over previous
import jax
import jax.numpy as jnp
from jax.experimental import pallas as pl
from jax.experimental.pallas import tpu as pltpu

H = 128
W = 128
N = H * W
C = 2048
NCH = N // C
HIDDEN = 64
NUM_COLORS = 10
LAYERS = 4
EPS = 1e-5


def _chunk_consts(s):
    idx = s + jax.lax.broadcasted_iota(jnp.int32, (C, 1), 0)
    r = idx // W
    c = idx - r * W
    one = jnp.float32(1.0)
    zero = jnp.float32(0.0)
    has_l = jnp.where(c > 0, one, zero)
    has_r = jnp.where(c < W - 1, one, zero)
    has_u = jnp.where(r > 0, one, zero)
    has_d = jnp.where(r < H - 1, one, zero)
    deg = 1.0 + has_l + has_r + has_u + has_d
    dinv = jax.lax.rsqrt(deg)
    rn = r.astype(jnp.float32) * (1.0 / (H - 1))
    cn = c.astype(jnp.float32) * (1.0 / (W - 1))
    return has_l, has_r, dinv, rn, cn


def _encoder_kernel(grids_ref, w_in_ref, b_in_ref, gcn_w_ref, gcn_b_ref,
                    ln_g_ref, ln_b_ref, w_out_ref, b_out_ref, out_ref,
                    x_buf, z_buf):
    zero = jnp.float32(0.0)

    def init_body(i, _):
        s = i * C
        _, _, _, rn, cn = _chunk_consts(s)
        g = grids_ref[0, pl.ds(s, C), :]
        x = jnp.zeros((C, HIDDEN), jnp.float32)
        for k in range(NUM_COLORS):
            x = x + jnp.where(g == k, 1.0, 0.0) * w_in_ref[k][None, :]
        x = x + rn * w_in_ref[NUM_COLORS][None, :]
        x = x + cn * w_in_ref[NUM_COLORS + 1][None, :]
        x_buf[pl.ds(s, C), :] = jnp.maximum(x + b_in_ref[...][None, :], zero)
        return 0

    jax.lax.fori_loop(0, NCH, init_body, 0)

    z_buf[0:W, :] = jnp.zeros((W, HIDDEN), jnp.float32)
    z_buf[W + N:, :] = jnp.zeros((W, HIDDEN), jnp.float32)

    for l in range(LAYERS):
        def z_body(i, _, l=l):
            s = i * C
            _, _, dinv, _, _ = _chunk_consts(s)
            xc = x_buf[pl.ds(s, C), :]
            z = jnp.dot(xc, gcn_w_ref[l], preferred_element_type=jnp.float32)
            z_buf[pl.ds(W + s, C), :] = z * dinv
            return 0

        jax.lax.fori_loop(0, NCH, z_body, 0)

        def agg_body(i, _, l=l):
            s = i * C
            has_l, has_r, dinv, _, _ = _chunk_consts(s)
            seg = z_buf[pl.ds(s, C + 2 * W), :]
            agg = seg[W:W + C]
            agg = agg + seg[0:C] + seg[2 * W:2 * W + C]
            agg = agg + has_l * seg[W - 1:W - 1 + C]
            agg = agg + has_r * seg[W + 1:W + 1 + C]
            y = agg * dinv + gcn_b_ref[l][None, :]
            mu = jnp.mean(y, axis=1, keepdims=True)
            d = y - mu
            var = jnp.mean(d * d, axis=1, keepdims=True)
            y = d * jax.lax.rsqrt(var + EPS) * ln_g_ref[l][None, :] \
                + ln_b_ref[l][None, :]
            x_buf[pl.ds(s, C), :] = jnp.maximum(y, zero) + x_buf[pl.ds(s, C), :]
            return 0

        jax.lax.fori_loop(0, NCH, agg_body, 0)

    def out_body(i, _):
        s = i * C
        xc = x_buf[pl.ds(s, C), :]
        out_ref[0, pl.ds(s, C), :] = \
            jnp.dot(xc, w_out_ref[...], preferred_element_type=jnp.float32) \
            + b_out_ref[...][None, :]
        return 0

    jax.lax.fori_loop(0, NCH, out_body, 0)


def kernel(grids, W_in, b_in, gcn_W, gcn_b, ln_g, ln_b, W_out, b_out):
    B = grids.shape[0]
    feat = W_out.shape[1]
    grids = grids.astype(jnp.float32).reshape(B, N, 1)
    full = lambda *shape: pl.BlockSpec(shape, lambda b: (0,) * len(shape))
    return pl.pallas_call(
        _encoder_kernel,
        grid=(B,),
        in_specs=[
            pl.BlockSpec((1, N, 1), lambda b: (b, 0, 0)),
            full(*W_in.shape),
            full(*b_in.shape),
            full(*gcn_W.shape),
            full(*gcn_b.shape),
            full(*ln_g.shape),
            full(*ln_b.shape),
            full(*W_out.shape),
            full(*b_out.shape),
        ],
        out_specs=pl.BlockSpec((1, N, feat), lambda b: (b, 0, 0)),
        out_shape=jax.ShapeDtypeStruct((B, N, feat), jnp.float32),
        scratch_shapes=[
            pltpu.VMEM((N, HIDDEN), jnp.float32),
            pltpu.VMEM((N + 2 * W, HIDDEN), jnp.float32),
        ],
    )(grids, W_in, b_in, gcn_W, gcn_b, ln_g, ln_b, W_out, b_out)


# speedup vs baseline: 10.5065x; 1.0956x over previous
import jax
import jax.numpy as jnp
from jax.experimental import pallas as pl
from jax.experimental.pallas import tpu as pltpu

H = 128
W = 128
N = H * W
C = 2048
NCH = N // C
HIDDEN = 64
NUM_COLORS = 10
FEAT_PACK = 16
LAYERS = 4
EPS = 1e-5


def _chunk_consts(s):
    idx = s + jax.lax.broadcasted_iota(jnp.int32, (C, 1), 0)
    r = idx // W
    c = idx - r * W
    one = jnp.float32(1.0)
    zero = jnp.float32(0.0)
    has_l = jnp.where(c > 0, one, zero)
    has_r = jnp.where(c < W - 1, one, zero)
    has_u = jnp.where(r > 0, one, zero)
    has_d = jnp.where(r < H - 1, one, zero)
    deg = 1.0 + has_l + has_r + has_u + has_d
    dinv = jax.lax.rsqrt(deg)
    rn = r.astype(jnp.float32) * (1.0 / (H - 1))
    cn = c.astype(jnp.float32) * (1.0 / (W - 1))
    return has_l, has_r, dinv, rn, cn


def _encoder_kernel(grids_ref, w_pack_ref, gcn_w_ref, gcn_b_ref,
                    ln_g_ref, ln_b_ref, w_out_ref, b_out_ref, out_ref,
                    x_buf, z_buf):
    zero = jnp.float32(0.0)

    def init_body(i, _):
        s = i * C
        _, _, _, rn, cn = _chunk_consts(s)
        g = grids_ref[0, pl.ds(s, C), :].astype(jnp.int32)
        lane = jax.lax.broadcasted_iota(jnp.int32, (C, FEAT_PACK), 1)
        feat = jnp.where(g == lane, 1.0, 0.0)
        feat = jnp.where(lane == NUM_COLORS, rn, feat)
        feat = jnp.where(lane == NUM_COLORS + 1, cn, feat)
        feat = jnp.where(lane == NUM_COLORS + 2, 1.0, feat)
        x = jnp.dot(feat, w_pack_ref[...], preferred_element_type=jnp.float32)
        x_buf[pl.ds(s, C), :] = jnp.maximum(x, zero)
        return 0

    jax.lax.fori_loop(0, NCH, init_body, 0)

    z_buf[0:W, :] = jnp.zeros((W, HIDDEN), jnp.float32)
    z_buf[W + N:, :] = jnp.zeros((W, HIDDEN), jnp.float32)

    for l in range(LAYERS):
        def z_body(i, _, l=l):
            s = i * C
            _, _, dinv, _, _ = _chunk_consts(s)
            xc = x_buf[pl.ds(s, C), :]
            z = jnp.dot(xc, gcn_w_ref[l], preferred_element_type=jnp.float32)
            z_buf[pl.ds(W + s, C), :] = z * dinv
            return 0

        jax.lax.fori_loop(0, NCH, z_body, 0)

        def agg_body(i, _, l=l):
            s = i * C
            has_l, has_r, dinv, _, _ = _chunk_consts(s)
            seg = z_buf[pl.ds(s, C + 2 * W), :]
            agg = seg[W:W + C]
            agg = agg + seg[0:C] + seg[2 * W:2 * W + C]
            agg = agg + has_l * seg[W - 1:W - 1 + C]
            agg = agg + has_r * seg[W + 1:W + 1 + C]
            y = agg * dinv + gcn_b_ref[l][None, :]
            mu = jnp.mean(y, axis=1, keepdims=True)
            d = y - mu
            var = jnp.mean(d * d, axis=1, keepdims=True)
            y = d * jax.lax.rsqrt(var + EPS) * ln_g_ref[l][None, :] \
                + ln_b_ref[l][None, :]
            x_buf[pl.ds(s, C), :] = jnp.maximum(y, zero) + x_buf[pl.ds(s, C), :]
            return 0

        jax.lax.fori_loop(0, NCH, agg_body, 0)

    def out_body(i, _):
        s = i * C
        xc = x_buf[pl.ds(s, C), :]
        out_ref[0, pl.ds(s, C), :] = \
            jnp.dot(xc, w_out_ref[...], preferred_element_type=jnp.float32) \
            + b_out_ref[...][None, :]
        return 0

    jax.lax.fori_loop(0, NCH, out_body, 0)


def kernel(grids, W_in, b_in, gcn_W, gcn_b, ln_g, ln_b, W_out, b_out):
    B = grids.shape[0]
    feat = W_out.shape[1]
    grids = grids.astype(jnp.int8).reshape(B, N, 1)
    w_pack = jnp.concatenate(
        [W_in, b_in[None, :],
         jnp.zeros((FEAT_PACK - W_in.shape[0] - 1, HIDDEN), jnp.float32)],
        axis=0)
    full = lambda *shape: pl.BlockSpec(shape, lambda b: (0,) * len(shape))
    return pl.pallas_call(
        _encoder_kernel,
        grid=(B,),
        in_specs=[
            pl.BlockSpec((1, N, 1), lambda b: (b, 0, 0)),
            full(*w_pack.shape),
            full(*gcn_W.shape),
            full(*gcn_b.shape),
            full(*ln_g.shape),
            full(*ln_b.shape),
            full(*W_out.shape),
            full(*b_out.shape),
        ],
        out_specs=pl.BlockSpec((1, N, feat), lambda b: (b, 0, 0)),
        out_shape=jax.ShapeDtypeStruct((B, N, feat), jnp.float32),
        scratch_shapes=[
            pltpu.VMEM((N, HIDDEN), jnp.float32),
            pltpu.VMEM((N + 2 * W, HIDDEN), jnp.float32),
        ],
        compiler_params=pltpu.CompilerParams(
            dimension_semantics=("parallel",)),
    )(grids, w_pack, gcn_W, gcn_b, ln_g, ln_b, W_out, b_out)


# speedup vs baseline: 24.0731x; 2.2912x over previous
import jax
import jax.numpy as jnp
from jax.experimental import pallas as pl
from jax.experimental.pallas import tpu as pltpu

H = 128
W = 128
N = H * W
C = 2048
NCH = N // C
HIDDEN = 64
NUM_COLORS = 10
FEAT_PACK = 16
LAYERS = 4
EPS = 1e-5


def _encoder_kernel(grids_ref, w_pack_ref, gcn_w_ref, gcn_b_ref,
                    ln_g_ref, ln_b_ref, w_out_ref, b_out_ref, out_ref,
                    x_buf, z_buf):
    zero = jnp.float32(0.0)
    one = jnp.float32(1.0)

    idx = jax.lax.broadcasted_iota(jnp.int32, (C, HIDDEN), 0)
    c = idx % W
    has_l = jnp.where(c > 0, one, zero)
    has_r = jnp.where(c < W - 1, one, zero)
    deg_mid = 3.0 + has_l + has_r
    dinv_mid = jax.lax.rsqrt(deg_mid)
    edge_row = jnp.where(idx < W, one, zero)
    edge_row_last = jnp.where(idx >= C - W, one, zero)
    dinv_first = jax.lax.rsqrt(deg_mid - edge_row)
    dinv_last = jax.lax.rsqrt(deg_mid - edge_row_last)

    cn16 = (jax.lax.broadcasted_iota(jnp.int32, (C, FEAT_PACK), 0) % W) \
        .astype(jnp.float32) * (1.0 / (W - 1))
    rstep16 = (jax.lax.broadcasted_iota(jnp.int32, (C, FEAT_PACK), 0) // W) \
        .astype(jnp.float32) * (1.0 / (H - 1))
    lane16 = jax.lax.broadcasted_iota(jnp.int32, (C, FEAT_PACK), 1)

    def init_body(i, _):
        s = i * C
        rn = rstep16 + (jnp.float32(C // W) * (1.0 / (H - 1))) \
            * i.astype(jnp.float32)
        g = grids_ref[0, pl.ds(s, C), :].astype(jnp.int32)
        feat = jnp.where(g == lane16, 1.0, 0.0)
        feat = jnp.where(lane16 == NUM_COLORS, rn, feat)
        feat = jnp.where(lane16 == NUM_COLORS + 1, cn16, feat)
        feat = jnp.where(lane16 == NUM_COLORS + 2, 1.0, feat)
        x = jnp.dot(feat, w_pack_ref[...], preferred_element_type=jnp.float32)
        x_buf[pl.ds(s, C), :] = jnp.maximum(x, zero)
        return 0

    jax.lax.fori_loop(0, NCH, init_body, 0)

    z_buf[0:W, :] = jnp.zeros((W, HIDDEN), jnp.float32)
    z_buf[W + N:, :] = jnp.zeros((W, HIDDEN), jnp.float32)

    def pick_dinv(i):
        d = jnp.where(i == 0, dinv_first, dinv_mid)
        return jnp.where(i == NCH - 1, dinv_last, d)

    for l in range(LAYERS):
        def z_body(i, _, l=l):
            s = i * C
            xc = x_buf[pl.ds(s, C), :]
            z = jnp.dot(xc, gcn_w_ref[l], preferred_element_type=jnp.float32)
            z_buf[pl.ds(W + s, C), :] = z * pick_dinv(i)
            return 0

        jax.lax.fori_loop(0, NCH, z_body, 0)

        def agg_body(i, _, l=l):
            s = i * C
            seg = z_buf[pl.ds(s, C + 2 * W), :]
            agg = seg[W:W + C]
            agg = agg + seg[0:C] + seg[2 * W:2 * W + C]
            agg = agg + has_l * seg[W - 1:W - 1 + C]
            agg = agg + has_r * seg[W + 1:W + 1 + C]
            y = agg * pick_dinv(i) + gcn_b_ref[l][None, :]
            mu = jnp.mean(y, axis=1, keepdims=True)
            d = y - mu
            var = jnp.mean(d * d, axis=1, keepdims=True)
            y = d * jax.lax.rsqrt(var + EPS) * ln_g_ref[l][None, :] \
                + ln_b_ref[l][None, :]
            x_buf[pl.ds(s, C), :] = jnp.maximum(y, zero) + x_buf[pl.ds(s, C), :]
            return 0

        jax.lax.fori_loop(0, NCH, agg_body, 0)

    def out_body(i, _):
        s = i * C
        xc = x_buf[pl.ds(s, C), :]
        out_ref[0, pl.ds(s, C), :] = \
            jnp.dot(xc, w_out_ref[...], preferred_element_type=jnp.float32) \
            + b_out_ref[...][None, :]
        return 0

    jax.lax.fori_loop(0, NCH, out_body, 0)


def kernel(grids, W_in, b_in, gcn_W, gcn_b, ln_g, ln_b, W_out, b_out):
    B = grids.shape[0]
    feat = W_out.shape[1]
    grids = grids.astype(jnp.int8).reshape(B, N, 1)
    w_pack = jnp.concatenate(
        [W_in, b_in[None, :],
         jnp.zeros((FEAT_PACK - W_in.shape[0] - 1, HIDDEN), jnp.float32)],
        axis=0)
    full = lambda *shape: pl.BlockSpec(shape, lambda b: (0,) * len(shape))
    return pl.pallas_call(
        _encoder_kernel,
        grid=(B,),
        in_specs=[
            pl.BlockSpec((1, N, 1), lambda b: (b, 0, 0)),
            full(*w_pack.shape),
            full(*gcn_W.shape),
            full(*gcn_b.shape),
            full(*ln_g.shape),
            full(*ln_b.shape),
            full(*W_out.shape),
            full(*b_out.shape),
        ],
        out_specs=pl.BlockSpec((1, N, feat), lambda b: (b, 0, 0)),
        out_shape=jax.ShapeDtypeStruct((B, N, feat), jnp.float32),
        scratch_shapes=[
            pltpu.VMEM((N, HIDDEN), jnp.float32),
            pltpu.VMEM((N + 2 * W, HIDDEN), jnp.float32),
        ],
        compiler_params=pltpu.CompilerParams(
            dimension_semantics=("parallel",)),
    )(grids, w_pack, gcn_W, gcn_b, ln_g, ln_b, W_out, b_out)


# speedup vs baseline: 31.2010x; 1.2961x over previous
import jax
import jax.numpy as jnp
from jax.experimental import pallas as pl
from jax.experimental.pallas import tpu as pltpu

H = 128
W = 128
N = H * W
C = 2048
NCH = N // C
HIDDEN = 64
HID2 = 2 * HIDDEN
NUM_COLORS = 10
FEAT_PACK = 16
FEAT2 = 2 * FEAT_PACK
LAYERS = 4
EPS = 1e-5


def _encoder_kernel(grids_ref, w_pack_ref, gcn_w_ref, gcn_b_ref,
                    ln_g_ref, ln_b_ref, w_out_ref, b_out_ref, ones_ref,
                    out_ref, x_buf, z_buf):
    zero = jnp.float32(0.0)
    one = jnp.float32(1.0)

    idx = jax.lax.broadcasted_iota(jnp.int32, (C, HID2), 0)
    c = idx % W
    has_l = jnp.where(c > 0, one, zero)
    has_r = jnp.where(c < W - 1, one, zero)
    deg_mid = 3.0 + has_l + has_r
    dinv_mid = jax.lax.rsqrt(deg_mid)
    edge_row = jnp.where(idx < W, one, zero)
    edge_row_last = jnp.where(idx >= C - W, one, zero)
    dinv_first = jax.lax.rsqrt(deg_mid - edge_row)
    dinv_last = jax.lax.rsqrt(deg_mid - edge_row_last)

    lane = jax.lax.broadcasted_iota(jnp.int32, (C, FEAT2), 1)
    l16 = lane % FEAT_PACK
    idx16 = jax.lax.broadcasted_iota(jnp.int32, (C, FEAT2), 0)
    cn16 = (idx16 % W).astype(jnp.float32) * (1.0 / (W - 1))
    rbase16 = (idx16 // W).astype(jnp.float32) * (1.0 / (H - 1))

    def init_body(i, _):
        s = i * C
        rn = rbase16 + (jnp.float32(C // W) * (1.0 / (H - 1))) \
            * i.astype(jnp.float32)
        g2 = grids_ref[0, pl.ds(s, C), :].astype(jnp.int32)
        gs = jnp.where(lane < FEAT_PACK, g2[:, 0:1], g2[:, 1:2])
        feat = jnp.where(gs == l16, 1.0, 0.0)
        feat = jnp.where(l16 == NUM_COLORS, rn, feat)
        feat = jnp.where(l16 == NUM_COLORS + 1, cn16, feat)
        feat = jnp.where(l16 == NUM_COLORS + 2, 1.0, feat)
        x = jnp.dot(feat, w_pack_ref[...], preferred_element_type=jnp.float32)
        x_buf[pl.ds(s, C), :] = jnp.maximum(x, zero)
        return 0

    jax.lax.fori_loop(0, NCH, init_body, 0)

    z_buf[0:W, :] = jnp.zeros((W, HID2), jnp.float32)
    z_buf[W + N:, :] = jnp.zeros((W, HID2), jnp.float32)

    def pick_dinv(i):
        d = jnp.where(i == 0, dinv_first, dinv_mid)
        return jnp.where(i == NCH - 1, dinv_last, d)

    for l in range(LAYERS):
        def z_body(i, _, l=l):
            s = i * C
            xc = x_buf[pl.ds(s, C), :]
            z = jnp.dot(xc, gcn_w_ref[l], preferred_element_type=jnp.float32)
            z_buf[pl.ds(W + s, C), :] = z * pick_dinv(i)
            return 0

        jax.lax.fori_loop(0, NCH, z_body, 0)

        def agg_body(i, _, l=l):
            s = i * C
            seg = z_buf[pl.ds(s, C + 2 * W), :]
            agg = seg[W:W + C]
            agg = agg + seg[0:C] + seg[2 * W:2 * W + C]
            agg = agg + has_l * seg[W - 1:W - 1 + C]
            agg = agg + has_r * seg[W + 1:W + 1 + C]
            d = agg * pick_dinv(i) + gcn_b_ref[l][None, :]
            var = jnp.dot(d * d, ones_ref[...],
                          preferred_element_type=jnp.float32)
            y = d * jax.lax.rsqrt(var + EPS) * ln_g_ref[l][None, :] \
                + ln_b_ref[l][None, :]
            x_buf[pl.ds(s, C), :] = jnp.maximum(y, zero) + x_buf[pl.ds(s, C), :]
            return 0

        jax.lax.fori_loop(0, NCH, agg_body, 0)

    def out_body(i, _):
        s = i * C
        xc = x_buf[pl.ds(s, C), :]
        out_ref[0, pl.ds(s, C), :] = \
            jnp.dot(xc, w_out_ref[...], preferred_element_type=jnp.float32) \
            + b_out_ref[...][None, :]
        return 0

    jax.lax.fori_loop(0, NCH, out_body, 0)


def _blockdiag2(m):
    r, c = m.shape[-2], m.shape[-1]
    z = jnp.zeros(m.shape[:-2] + (2 * r, 2 * c), m.dtype)
    return z.at[..., :r, :c].set(m).at[..., r:, c:].set(m)


def kernel(grids, W_in, b_in, gcn_W, gcn_b, ln_g, ln_b, W_out, b_out):
    B = grids.shape[0]
    B2 = B // 2
    feat = W_out.shape[1]
    grids2 = grids.astype(jnp.int8).reshape(B2, 2, N).transpose(0, 2, 1)

    ctr = jnp.eye(HIDDEN, dtype=jnp.float32) - 1.0 / HIDDEN
    gcn_Wc = jnp.matmul(gcn_W, ctr)
    gcn_bc = jnp.matmul(gcn_b, ctr)

    w_pack1 = jnp.concatenate(
        [W_in, b_in[None, :],
         jnp.zeros((FEAT_PACK - W_in.shape[0] - 1, HIDDEN), jnp.float32)],
        axis=0)
    w_pack = _blockdiag2(w_pack1)
    gcn_W2 = _blockdiag2(gcn_Wc)
    w_out2 = _blockdiag2(W_out)
    ones_blk = _blockdiag2(jnp.full((HIDDEN, HIDDEN), 1.0 / HIDDEN,
                                    jnp.float32))
    dup = lambda v: jnp.concatenate([v, v], axis=-1)
    gcn_b2 = dup(gcn_bc)
    ln_g2 = dup(ln_g)
    ln_b2 = dup(ln_b)
    b_out2 = dup(b_out)

    full = lambda *shape: pl.BlockSpec(shape, lambda b: (0,) * len(shape))
    out = pl.pallas_call(
        _encoder_kernel,
        grid=(B2,),
        in_specs=[
            pl.BlockSpec((1, N, 2), lambda b: (b, 0, 0)),
            full(*w_pack.shape),
            full(*gcn_W2.shape),
            full(*gcn_b2.shape),
            full(*ln_g2.shape),
            full(*ln_b2.shape),
            full(*w_out2.shape),
            full(*b_out2.shape),
            full(*ones_blk.shape),
        ],
        out_specs=pl.BlockSpec((1, N, HID2), lambda b: (b, 0, 0)),
        out_shape=jax.ShapeDtypeStruct((B2, N, HID2), jnp.float32),
        scratch_shapes=[
            pltpu.VMEM((N, HID2), jnp.float32),
            pltpu.VMEM((N + 2 * W, HID2), jnp.float32),
        ],
        compiler_params=pltpu.CompilerParams(
            dimension_semantics=("parallel",)),
    )(grids2, w_pack, gcn_W2, gcn_b2, ln_g2, ln_b2, w_out2, b_out2, ones_blk)
    return out.reshape(B2, N, 2, feat).transpose(0, 2, 1, 3).reshape(B, N, feat)
